# baseline (speedup 1.0000x reference)
import jax
import jax.numpy as jnp
from jax.experimental import pallas as pl


def _heads_body(x0, w0, b0, x1, w1, b1, x2, w2, b2, o0, o1, o2):
    o0[0] = jnp.dot(w0[...], x0[0], preferred_element_type=jnp.float32) + b0[...]
    o1[0] = jnp.dot(w1[...], x1[0], preferred_element_type=jnp.float32) + b1[...]
    o2[0] = jnp.dot(w2[...], x2[0], preferred_element_type=jnp.float32) + b2[...]


def kernel(feats_0, feats_1, feats_2, W0, b0, W1, b1, W2, b2):
    B = feats_0.shape[0]
    shapes = [feats_0.shape, feats_1.shape, feats_2.shape]
    xs = [f.reshape(f.shape[0], f.shape[1], f.shape[2] * f.shape[3])
          for f in (feats_0, feats_1, feats_2)]
    ws = [W0, W1, W2]
    bs = [b.reshape(-1, 1) for b in (b0, b1, b2)]
    out_dim = W0.shape[0]

    def feat_spec(x):
        return pl.BlockSpec((1, x.shape[1], x.shape[2]), lambda b: (b, 0, 0))

    def full_spec(a):
        return pl.BlockSpec(a.shape, lambda b: (0,) * a.ndim)

    in_specs = []
    for x, w, bia in zip(xs, ws, bs):
        in_specs.append(feat_spec(x))
        in_specs.append(full_spec(w))
        in_specs.append(full_spec(bia))

    out_shapes = [jax.ShapeDtypeStruct((B, out_dim, x.shape[2]), jnp.float32)
                  for x in xs]
    out_specs = [pl.BlockSpec((1, out_dim, x.shape[2]), lambda b: (b, 0, 0))
                 for x in xs]

    outs = pl.pallas_call(
        _heads_body,
        grid=(B,),
        in_specs=in_specs,
        out_specs=out_specs,
        out_shape=out_shapes,
    )(xs[0], ws[0], bs[0], xs[1], ws[1], bs[1], xs[2], ws[2], bs[2])

    return tuple(
        o.reshape(s[0], out_dim, s[2], s[3]) for o, s in zip(outs, shapes)
    )
